# baseline (speedup 1.0000x reference)
import dataclasses
import functools

import jax
import jax.numpy as jnp
from jax import lax
from jax.experimental import pallas as pl
from jax.experimental.pallas import tpu as pltpu
from jax.experimental.pallas import tpu_sc as plsc

_NUM_CORES = 2
_NUM_SUBCORES = 16
_NW = _NUM_CORES * _NUM_SUBCORES
_LANES = 16
_IDX_CHUNK = 128
_ROW = 8


@functools.lru_cache(maxsize=None)
def _build_sc_lerp_gather(n_start: int, nc: int, two: int, t_total: int):
    assert t_total % _NW == 0 and (nc * two) % _ROW == 0
    w = t_total // _NW
    nq = w // _IDX_CHUNK
    nrows = nc * two // _ROW
    assert w % _IDX_CHUNK == 0
    mesh = plsc.VectorSubcoreMesh(core_axis_name="c", subcore_axis_name="s")
    cp = pltpu.CompilerParams()
    for _f, _v in (("needs_layout_passes", False),
                   ("use_tc_tiling_on_sc", False)):
        if _f in pltpu.CompilerParams.__dataclass_fields__:
            cp = dataclasses.replace(cp, **{_f: _v})

    @functools.partial(
        pl.kernel,
        out_type=jax.ShapeDtypeStruct((n_start, t_total, two), jnp.float32),
        mesh=mesh,
        compiler_params=cp,
        scratch_types=[
            pltpu.VMEM((w,), jnp.float32),
            pltpu.VMEM((w,), jnp.float32),
            pltpu.VMEM((w,), jnp.int32),
            pltpu.VMEM((nq, _IDX_CHUNK), jnp.int32),
            pltpu.VMEM((nq, _IDX_CHUNK), jnp.int32),
            pltpu.VMEM((2 * w, _ROW), jnp.float32),
            pltpu.VMEM((w, two), jnp.float32),
            pltpu.SemaphoreType.DMA,
        ],
    )
    def sc_kernel(cp_hbm, ts_hbm, out_hbm,
                  ts_v, frac_v, off_v, idx0_v, idx1_v, buf_v, o_v, sem):
        wid = lax.axis_index("s") * _NUM_CORES + lax.axis_index("c")
        t0 = wid * w
        pltpu.sync_copy(ts_hbm.at[pl.ds(t0, w)], ts_v)

        @pl.loop(0, w, step=_LANES)
        def _(i):
            tv = ts_v[pl.ds(i, _LANES)]
            idx = (tv * float(nc - 2)).astype(jnp.int32)
            frac_v[pl.ds(i, _LANES)] = (
                tv * float(nc - 1) - idx.astype(jnp.float32))
            off_v[pl.ds(i, _LANES)] = (idx & 3) * 2
            r0 = lax.shift_right_logical(idx, 2)
            q = i // _IDX_CHUNK
            o = i % _IDX_CHUNK
            idx0_v[q, pl.ds(o, _LANES)] = r0
            idx1_v[q, pl.ds(o, _LANES)] = jnp.minimum(r0 + 1, nrows - 1)

        lane = lax.iota(jnp.int32, _LANES)
        rowsel = lax.shift_right_logical(lane, 1)
        colsel = lax.bitwise_and(lane, 1)

        @pl.loop(0, n_start)
        def _(s):
            cp_s = cp_hbm.at[s]
            copies = []
            for q in range(nq):
                copies.append(pltpu.async_copy(
                    cp_s.at[idx0_v.at[q]],
                    buf_v.at[pl.ds(q * _IDX_CHUNK, _IDX_CHUNK)], sem))
                copies.append(pltpu.async_copy(
                    cp_s.at[idx1_v.at[q]],
                    buf_v.at[pl.ds(w + q * _IDX_CHUNK, _IDX_CHUNK)], sem))
            for c in copies:
                c.wait()

            @pl.loop(0, w // (_LANES // two))
            def _(k8):
                rows = rowsel + k8 * (_LANES // two)
                offv = plsc.load_gather(off_v, [rows])
                col_l = offv + colsel
                lv = plsc.load_gather(buf_v, [rows, col_l])
                cross = offv == (_ROW - 2)
                row_r = rows + jnp.where(cross, w, 0)
                col_r = jnp.where(cross, colsel, col_l + 2)
                rv = plsc.load_gather(buf_v, [row_r, col_r])
                fv = plsc.load_gather(frac_v, [rows])
                ov = (1.0 - fv) * lv + fv * rv
                plsc.store_scatter(o_v, [rows, colsel], ov)

            pltpu.sync_copy(o_v, out_hbm.at[s, pl.ds(t0, w)])

    return sc_kernel


def kernel(timestamps, control_points):
    n_start, nc, two = control_points.shape
    t_total = timestamps.shape[0]
    rows_view = control_points.reshape(n_start, (nc * two) // _ROW, _ROW)
    sc_kernel = _build_sc_lerp_gather(n_start, nc, two, t_total)
    return sc_kernel(rows_view, timestamps)
